# baseline (speedup 1.0000x reference)
import functools

import jax
import jax.numpy as jnp
from jax import lax
from jax.experimental import pallas as pl
from jax.experimental.pallas import tpu as pltpu
from jax.experimental.pallas import tpu_sc as plsc

DIM = 32
BATCH = 4096
HIST = 200
NUM_EMB = 1_000_000
NUM_CORES = 2
NUM_SUBCORES = 16
NW = NUM_CORES * NUM_SUBCORES
MBLK = BATCH // 128
JOBS = HIST * MBLK
JOB_PER_W = JOBS // NW
OUT_WORDS = BATCH * HIST * DIM
TILES = OUT_WORDS // 1024
CHUNK_T = 32
NCHUNK = TILES // CHUNK_T
CHUNK_PER_W = NCHUNK // NW

_mesh = plsc.VectorSubcoreMesh(core_axis_name="c", subcore_axis_name="s")


@functools.partial(
    pl.kernel,
    mesh=_mesh,
    out_type=jax.ShapeDtypeStruct((OUT_WORDS,), jnp.float32),
    scratch_types=[
        pltpu.VMEM((128,), jnp.int32),
        pltpu.VMEM((128, DIM), jnp.float32),
        pltpu.VMEM((4096,), jnp.float32),
        pltpu.SemaphoreType.DMA,
    ],
    compiler_params=pltpu.CompilerParams(
        use_tc_tiling_on_sc=False, needs_layout_passes=False),
)
def _gather(tok, table, out, idxv, gbuf, arena, sem):
    wid = lax.axis_index("s") * NUM_CORES + lax.axis_index("c")
    iota = lax.iota(jnp.int32, 16)

    def job(t, carry):
        j = t * NW + wid
        h = j // MBLK
        m = j % MBLK
        pltpu.sync_copy(tok.at[pl.ds(h * BATCH + m * 128, 128)], idxv)
        pltpu.async_copy(table.at[idxv], gbuf, sem).wait()
        for g in range(8):
            rows = iota + g * 16
            for d in range(DIM):
                v = plsc.load_gather(gbuf, [rows, jnp.full((16,), d, jnp.int32)])
                arena[pl.ds((d // 8) * 1024 + (d % 8) * 128 + g * 16, 16)] = v
        base = (h * 128 + m) * 1024
        for k in range(4):
            pltpu.sync_copy(arena.at[pl.ds(k * 1024, 1024)],
                            out.at[pl.ds(base + k * 32 * 1024, 1024)])
        return carry

    lax.fori_loop(0, JOB_PER_W, job, 0)


@functools.partial(
    pl.kernel,
    mesh=_mesh,
    out_type=jax.ShapeDtypeStruct((HIST, DIM, BATCH), jnp.float32),
    scratch_types=[
        pltpu.VMEM((1, CHUNK_T * 8, 128), jnp.float32),
    ],
    compiler_params=pltpu.CompilerParams(use_tc_tiling_on_sc=True),
)
def _retile(lin, out, buf):
    wid = lax.axis_index("s") * NUM_CORES + lax.axis_index("c")

    def chunk(t, carry):
        r = t * NW + wid
        h = r // 4
        k = r % 4
        pltpu.sync_copy(lin.at[:, pl.ds(r * CHUNK_T * 8, CHUNK_T * 8), :], buf)
        for m in range(CHUNK_T):
            pltpu.sync_copy(
                buf.at[:, pl.ds(m * 8, 8), :],
                out.at[pl.ds(h, 1), pl.ds(k * 8, 8), pl.ds(m * 128, 128)])
        return carry

    lax.fori_loop(0, CHUNK_PER_W, chunk, 0)


def kernel(tokens, table):
    tok_flat = tokens.astype(jnp.int32).T.reshape(HIST * BATCH)
    lin = _gather(tok_flat, table)
    lin3 = lin.reshape(1, OUT_WORDS // 128, 128)
    out3 = _retile(lin3)
    return jnp.transpose(out3, (2, 0, 1))


# speedup vs baseline: 1.8747x; 1.8747x over previous
import functools

import jax
import jax.numpy as jnp
from jax import lax
from jax.experimental import pallas as pl
from jax.experimental.pallas import tpu as pltpu
from jax.experimental.pallas import tpu_sc as plsc

DIM = 32
BATCH = 4096
HIST = 200
NUM_EMB = 1_000_000
NUM_CORES = 2
NUM_SUBCORES = 16
NW = NUM_CORES * NUM_SUBCORES
MBLK = BATCH // 128
JOBS = HIST * MBLK
JOB_PER_W = JOBS // NW
OUT_WORDS = BATCH * HIST * DIM
TILES = OUT_WORDS // 1024
CHUNK_T = 32
NCHUNK = TILES // CHUNK_T
CHUNK_PER_W = NCHUNK // NW

_mesh = plsc.VectorSubcoreMesh(core_axis_name="c", subcore_axis_name="s")


SJ_TOK = 512
SJ = HIST * (BATCH // SJ_TOK)
SJ_PER_W = SJ // NW
SJ_MG = BATCH // SJ_TOK


@functools.partial(
    pl.kernel,
    mesh=_mesh,
    out_type=jax.ShapeDtypeStruct((OUT_WORDS // 128, 128), jnp.float32),
    scratch_types=[
        pltpu.VMEM((SJ_TOK,), jnp.int32),
        pltpu.VMEM((SJ_TOK,), jnp.int32),
        pltpu.VMEM((SJ_TOK, DIM), jnp.float32),
        pltpu.VMEM((SJ_TOK, DIM), jnp.float32),
        pltpu.VMEM((128, 128), jnp.float32),
        pltpu.VMEM((128, 128), jnp.float32),
        pltpu.SemaphoreType.DMA,
        pltpu.SemaphoreType.DMA,
        pltpu.SemaphoreType.DMA,
        pltpu.SemaphoreType.DMA,
        pltpu.SemaphoreType.DMA,
        pltpu.SemaphoreType.DMA,
    ],
    compiler_params=pltpu.CompilerParams(
        use_tc_tiling_on_sc=False, needs_layout_passes=False),
)
def _gather(tok, table, out, i0, i1, g0, g1, a0, a1,
            is0, is1, gs0, gs1, os0, os1):
    wid = lax.axis_index("s") * NUM_CORES + lax.axis_index("c")
    iota = lax.iota(jnp.int32, 16)
    idxb, gb, ab = (i0, i1), (g0, g1), (a0, a1)
    isem, gsem, osem = (is0, is1), (gs0, gs1), (os0, os1)

    def sj_of(t):
        return t * NW + wid

    def idx_start(t, b):
        s = sj_of(t)
        h = s // SJ_MG
        mg = s % SJ_MG
        pltpu.async_copy(tok.at[pl.ds(h * BATCH + mg * SJ_TOK, SJ_TOK)],
                         idxb[b], isem[b])

    def idx_wait(b):
        pltpu.make_async_copy(tok.at[pl.ds(0, SJ_TOK)], idxb[b],
                              isem[b]).wait()

    def gather_start(b):
        pltpu.async_copy(table.at[idxb[b]], gb[b], gsem[b])

    def gather_wait(b):
        pltpu.make_async_copy(table.at[idxb[b]], gb[b], gsem[b]).wait()

    def transpose(b):
        def drow(d0, carry):
            t_vec = (d0 + iota) & 31
            rows_sc = [t_vec + ml * 32 for ml in range(4)]
            for j0 in range(0, SJ_TOK, 16):
                v = plsc.load_gather(gb[b], [iota + j0, t_vec])
                plsc.store_scatter(ab[b],
                                   [rows_sc[j0 // 128], iota + (j0 % 128)], v)
            return carry

        lax.fori_loop(0, DIM, drow, 0)

    def out_start(t, b):
        s = sj_of(t)
        h = s // SJ_MG
        mg = s % SJ_MG
        for ml in range(4):
            for k in range(4):
                trow = (h * 128 + k * 32 + mg * 4 + ml) * 8
                pltpu.async_copy(ab[b].at[pl.ds(ml * 32 + k * 8, 8), :],
                                 out.at[pl.ds(trow, 8)], osem[b])

    def out_wait(b):
        for _ in range(16):
            pltpu.make_async_copy(ab[b].at[pl.ds(0, 8), :],
                                  out.at[pl.ds(0, 8)], osem[b]).wait()

    idx_start(0, 0)
    idx_start(1, 1)
    idx_wait(0)
    gather_start(0)

    def pair(p, carry):
        for i in range(2):
            t = p * 2 + i
            b = i
            gather_wait(b)

            @pl.when(t + 2 < SJ_PER_W)
            def _():
                idx_start(t + 2, b)

            @pl.when(t + 1 < SJ_PER_W)
            def _():
                idx_wait(1 - b)
                gather_start(1 - b)

            @pl.when(t >= 2)
            def _():
                out_wait(b)

            transpose(b)
            out_start(t, b)
        return carry

    lax.fori_loop(0, SJ_PER_W // 2, pair, 0)
    out_wait(0)
    out_wait(1)


@functools.partial(
    pl.kernel,
    mesh=_mesh,
    out_type=jax.ShapeDtypeStruct((HIST, DIM, BATCH), jnp.float32),
    scratch_types=[
        pltpu.VMEM((1, CHUNK_T * 8, 128), jnp.float32),
    ],
    compiler_params=pltpu.CompilerParams(use_tc_tiling_on_sc=True),
)
def _retile(lin, out, buf):
    wid = lax.axis_index("s") * NUM_CORES + lax.axis_index("c")

    def chunk(t, carry):
        r = t * NW + wid
        h = r // 4
        k = r % 4
        pltpu.sync_copy(lin.at[:, pl.ds(r * CHUNK_T * 8, CHUNK_T * 8), :], buf)
        for m in range(CHUNK_T):
            pltpu.sync_copy(
                buf.at[:, pl.ds(m * 8, 8), :],
                out.at[pl.ds(h, 1), pl.ds(k * 8, 8), pl.ds(m * 128, 128)])
        return carry

    lax.fori_loop(0, CHUNK_PER_W, chunk, 0)


def kernel(tokens, table):
    tok_flat = tokens.astype(jnp.int32).T.reshape(HIST * BATCH)
    lin = _gather(tok_flat, table)
    lin3 = lin.reshape(1, OUT_WORDS // 128, 128)
    out3 = _retile(lin3)
    return jnp.transpose(out3, (2, 0, 1))
